# baseline (speedup 1.0000x reference)
import jax
import jax.numpy as jnp
from jax import lax
from jax.experimental import pallas as pl
from jax.experimental.pallas import tpu as pltpu
from jax.experimental.pallas import tpu_sc as plsc

N_TAB = 26
B = 4096
V = 100000
D = 32
L = 20
NC, NS = 2, 16
NW = NC * NS
BPW = B // NW
CB = 16
NCH = BPW // CB
HB = CB // 2
RPT = HB * L
NTASK = 2 * N_TAB
NBUF = 13
OUTW = N_TAB * D


def _body(*refs):
    idx_refs = refs[:N_TAB]
    tab_refs = refs[N_TAB:2 * N_TAB]
    out_ref = refs[2 * N_TAB]
    scratch = refs[2 * N_TAB + 1:]
    idx_bufs = scratch[:NBUF]
    rows_bufs = scratch[NBUF:2 * NBUF]
    stage_v = scratch[2 * NBUF]
    sems = scratch[2 * NBUF + 1:]
    wid = lax.axis_index("s") * NC + lax.axis_index("c")
    bbase = wid * BPW

    def start_gather(c, task):
        t, h = task // 2, task % 2
        buf = task % NBUF
        ibase = (bbase + c * CB) * L + h * RPT
        pltpu.sync_copy(idx_refs[t].at[pl.ds(ibase, RPT)], idx_bufs[buf])
        for k in range(RPT // 16):
            idx_vec = idx_bufs[buf][pl.ds(k * 16, 16)]
            pltpu.async_copy(tab_refs[t].at[idx_vec],
                             rows_bufs[buf].at[pl.ds(k * 16, 16)],
                             sems[buf])

    def wait_gather(task):
        buf = task % NBUF
        pltpu.make_async_copy(tab_refs[0].at[pl.ds(0, RPT)],
                              rows_bufs[buf], sems[buf]).wait()

    for j in range(NBUF):
        start_gather(0, j)

    @pl.loop(0, NCH)
    def _chunk(c):
        for task in range(NTASK):
            t, h = task // 2, task % 2
            wait_gather(task)
            rows_v = rows_bufs[task % NBUF]

            @pl.loop(0, HB)
            def _compute(b):
                r0 = b * L
                a0 = rows_v[r0, pl.ds(0, 16)]
                a1 = rows_v[r0, pl.ds(16, 16)]
                for l in range(1, L):
                    a0 = a0 + rows_v[r0 + l, pl.ds(0, 16)]
                    a1 = a1 + rows_v[r0 + l, pl.ds(16, 16)]
                o0 = (h * HB + b) * OUTW + t * D
                stage_v[pl.ds(o0, 16)] = a0
                stage_v[pl.ds(o0 + 16, 16)] = a1

            nxt = task + NBUF
            if nxt < NTASK:
                start_gather(c, nxt)
            else:
                @pl.when(c + 1 < NCH)
                def _prefetch():
                    start_gather(c + 1, nxt - NTASK)

        pltpu.sync_copy(stage_v,
                        out_ref.at[pl.ds((bbase + c * CB) * OUTW, CB * OUTW)])


def _feature_layer(idx_flat, tables):
    mesh = plsc.VectorSubcoreMesh(core_axis_name="c", subcore_axis_name="s")
    scratch = ([pltpu.VMEM((RPT,), jnp.int32) for _ in range(NBUF)]
               + [pltpu.VMEM((RPT, D), jnp.float32) for _ in range(NBUF)]
               + [pltpu.VMEM((CB * OUTW,), jnp.float32)]
               + [pltpu.SemaphoreType.DMA for _ in range(NBUF)])
    return pl.kernel(
        _body,
        out_type=jax.ShapeDtypeStruct((B * OUTW,), jnp.float32),
        mesh=mesh,
        compiler_params=pltpu.CompilerParams(use_tc_tiling_on_sc=False),
        scratch_types=scratch,
    )(*idx_flat, *tables)


def kernel(f0, f1, f2, f3, f4, f5, f6, f7, f8, f9, f10, f11, f12, f13,
           f14, f15, f16, f17, f18, f19, f20, f21, f22, f23, f24, f25,
           table_0, table_1, table_2, table_3, table_4, table_5, table_6,
           table_7, table_8, table_9, table_10, table_11, table_12,
           table_13, table_14, table_15, table_16, table_17, table_18,
           table_19, table_20, table_21, table_22, table_23, table_24,
           table_25, dense_float, dense_array):
    fs = [f0, f1, f2, f3, f4, f5, f6, f7, f8, f9, f10, f11, f12, f13,
          f14, f15, f16, f17, f18, f19, f20, f21, f22, f23, f24, f25]
    tables = [table_0, table_1, table_2, table_3, table_4, table_5,
              table_6, table_7, table_8, table_9, table_10, table_11,
              table_12, table_13, table_14, table_15, table_16, table_17,
              table_18, table_19, table_20, table_21, table_22, table_23,
              table_24, table_25]
    idx_flat = [f.reshape(-1) for f in fs]
    emb = _feature_layer(idx_flat, tables).reshape(B, OUTW)
    return jnp.concatenate([emb, dense_float, dense_array], axis=-1)


# speedup vs baseline: 1.1225x; 1.1225x over previous
import jax
import jax.numpy as jnp
from jax import lax
from jax.experimental import pallas as pl
from jax.experimental.pallas import tpu as pltpu
from jax.experimental.pallas import tpu_sc as plsc

N_TAB = 26
B = 4096
V = 100000
D = 32
L = 20
NC, NS = 2, 16
NW = NC * NS
BPW = B // NW
CB = 32
NCH = BPW // CB
RPC = CB * L
OUTW = N_TAB * D


def _body(*refs):
    idx_refs = refs[:N_TAB]
    tab_refs = refs[N_TAB:2 * N_TAB]
    out_ref = refs[2 * N_TAB]
    idx_v0, idx_v1, rows_v0, rows_v1, stage_v, sem0, sem1 = refs[2 * N_TAB + 1:]
    idx_bufs = (idx_v0, idx_v1)
    rows_bufs = (rows_v0, rows_v1)
    sems = (sem0, sem1)
    wid = lax.axis_index("s") * NC + lax.axis_index("c")
    bbase = wid * BPW

    @pl.loop(0, NCH)
    def _chunk(c):
        ibase = (bbase + c * CB) * L

        def start_gather(t):
            buf = t % 2
            pltpu.sync_copy(idx_refs[t].at[pl.ds(ibase, RPC)],
                            idx_bufs[buf])
            return pltpu.async_copy(tab_refs[t].at[idx_bufs[buf]],
                                    rows_bufs[buf], sems[buf])

        pending = start_gather(0)
        for t in range(N_TAB):
            nxt = start_gather(t + 1) if t + 1 < N_TAB else None
            pending.wait()
            pending = nxt
            rows_v = rows_bufs[t % 2]

            @pl.loop(0, CB)
            def _compute(b):
                r0 = b * L
                a0 = rows_v[r0, pl.ds(0, 16)]
                a1 = rows_v[r0, pl.ds(16, 16)]
                for l in range(1, L):
                    a0 = a0 + rows_v[r0 + l, pl.ds(0, 16)]
                    a1 = a1 + rows_v[r0 + l, pl.ds(16, 16)]
                o0 = b * OUTW + t * D
                stage_v[pl.ds(o0, 16)] = a0
                stage_v[pl.ds(o0 + 16, 16)] = a1

        pltpu.sync_copy(stage_v,
                        out_ref.at[pl.ds((bbase + c * CB) * OUTW, CB * OUTW)])


def _feature_layer(idx_flat, tables):
    mesh = plsc.VectorSubcoreMesh(core_axis_name="c", subcore_axis_name="s")
    return pl.kernel(
        _body,
        out_type=jax.ShapeDtypeStruct((B * OUTW,), jnp.float32),
        mesh=mesh,
        compiler_params=pltpu.CompilerParams(use_tc_tiling_on_sc=False),
        scratch_types=[
            pltpu.VMEM((RPC,), jnp.int32),
            pltpu.VMEM((RPC,), jnp.int32),
            pltpu.VMEM((RPC, D), jnp.float32),
            pltpu.VMEM((RPC, D), jnp.float32),
            pltpu.VMEM((CB * OUTW,), jnp.float32),
            pltpu.SemaphoreType.DMA,
            pltpu.SemaphoreType.DMA,
        ],
    )(*idx_flat, *tables)


def kernel(f0, f1, f2, f3, f4, f5, f6, f7, f8, f9, f10, f11, f12, f13,
           f14, f15, f16, f17, f18, f19, f20, f21, f22, f23, f24, f25,
           table_0, table_1, table_2, table_3, table_4, table_5, table_6,
           table_7, table_8, table_9, table_10, table_11, table_12,
           table_13, table_14, table_15, table_16, table_17, table_18,
           table_19, table_20, table_21, table_22, table_23, table_24,
           table_25, dense_float, dense_array):
    fs = [f0, f1, f2, f3, f4, f5, f6, f7, f8, f9, f10, f11, f12, f13,
          f14, f15, f16, f17, f18, f19, f20, f21, f22, f23, f24, f25]
    tables = [table_0, table_1, table_2, table_3, table_4, table_5,
              table_6, table_7, table_8, table_9, table_10, table_11,
              table_12, table_13, table_14, table_15, table_16, table_17,
              table_18, table_19, table_20, table_21, table_22, table_23,
              table_24, table_25]
    idx_flat = [f.reshape(-1) for f in fs]
    emb = _feature_layer(idx_flat, tables).reshape(B, OUTW)
    return jnp.concatenate([emb, dense_float, dense_array], axis=-1)
